# Initial kernel scaffold; baseline (speedup 1.0000x reference)
import jax
import jax.numpy as jnp
from jax.experimental import pallas as pl


def kernel(ent_emb, rel_emb, ent_mask, rel_mask, question_emb, tok_W, tok_b, key_W, key_b, comp_W, comp_b, kg_W, kg_b, q_W, q_b, k_W, k_b, v_W, v_b, edge_index, rel_id):
    raise NotImplementedError("write your pallas kernel here")


# speedup vs baseline: 1.0126x; 1.0126x over previous
import jax
import jax.numpy as jnp
from jax.experimental import pallas as pl

N_ENT_ = 1968
H_ = 128
L_ = 16
E_ = 8192
B_ = 4
KSEL_ = int(N_ENT_ * 0.1)
TOPK_ = 2


def _bfdot(a, b, dims):
    return jax.lax.dot_general(a.astype(jnp.bfloat16), b.astype(jnp.bfloat16),
                               dims, preferred_element_type=jnp.float32)


def _tail_body(x_ref, attw_ref, idx_ref, vW_ref, vb_ref, kgW_ref, kgb_ref,
               out_ref):
    x = x_ref[0]
    V = _bfdot(x, vW_ref[...], (((1,), (1,)), ((), ()))) + vb_ref[0][None, :]
    idx = idx_ref[0, 0]
    cols = jax.lax.broadcasted_iota(jnp.int32, (KSEL_, N_ENT_), 1)
    onehot = (cols == idx[:, None]).astype(jnp.float32)
    sel_attw = _bfdot(onehot, attw_ref[0], (((1,), (0,)), ((), ())))
    sel_att = _bfdot(sel_attw, V, (((1,), (0,)), ((), ())))
    out = _bfdot(sel_att, kgW_ref[...], (((1,), (1,)), ((), ())))
    out_ref[0] = jnp.tanh(out + kgb_ref[0][None, :])


def _pallas_tail(x, attw, topk_idx, v_W, v_b, kg_W, kg_b):
    return pl.pallas_call(
        _tail_body,
        grid=(B_,),
        in_specs=[
            pl.BlockSpec((1, N_ENT_, H_), lambda b: (b, 0, 0)),
            pl.BlockSpec((1, N_ENT_, N_ENT_), lambda b: (b, 0, 0)),
            pl.BlockSpec((1, 1, KSEL_), lambda b: (b, 0, 0)),
            pl.BlockSpec((H_, H_), lambda b: (0, 0)),
            pl.BlockSpec((1, H_), lambda b: (0, 0)),
            pl.BlockSpec((H_, H_), lambda b: (0, 0)),
            pl.BlockSpec((1, H_), lambda b: (0, 0)),
        ],
        out_specs=pl.BlockSpec((1, KSEL_, H_), lambda b: (b, 0, 0)),
        out_shape=jax.ShapeDtypeStruct((B_, KSEL_, H_), jnp.float32),
    )(x, attw, topk_idx.reshape(B_, 1, KSEL_), v_W, v_b.reshape(1, H_),
      kg_W, kg_b.reshape(1, H_))


def _topk_out_edge_mask(src, w, k):
    n_e = w.shape[0]
    w = jax.lax.stop_gradient(w)
    order = jnp.lexsort((-w, src))
    s_sorted = src[order]
    seg_start = jnp.searchsorted(s_sorted, s_sorted, side='left')
    pos = jnp.arange(n_e) - seg_start
    keep_sorted = pos < k
    keep = jnp.zeros((n_e,), dtype=bool).at[order].set(keep_sorted)
    return keep


def kernel(ent_emb, rel_emb, ent_mask, rel_mask, question_emb, tok_W, tok_b,
           key_W, key_b, comp_W, comp_b, kg_W, kg_b, q_W, q_b, k_W, k_b,
           v_W, v_b, edge_index, rel_id):
    f32 = jnp.float32
    src = edge_index[0]
    dst = edge_index[1]
    e_emb = ent_emb[src] * rel_emb[rel_id]
    e_mask = ent_mask[src] + rel_mask[rel_id]
    comp_emb = e_emb * ent_emb[dst]
    query = question_emb[:, None, :, :]
    query_emb = query @ tok_W.T + tok_b
    key_emb = comp_emb @ key_W.T + key_b
    key_emb = jnp.transpose(key_emb, (0, 2, 1))
    scale = (H_ / 2.0) ** (-0.5)
    weight = scale * jnp.matmul(query_emb, key_emb)
    kg_mask = e_mask[None, :, None, :]
    nz = (kg_mask != 0).astype(f32)
    eps = jnp.sum(nz, axis=3)
    eps = jnp.where(eps == 0, jnp.ones_like(eps), eps)
    weight = jnp.sum(weight * kg_mask, axis=3) / eps
    weight = jnp.mean(weight, axis=-1)
    m_ = jnp.max(weight, axis=-1, keepdims=True)
    e_ = jnp.exp(weight - m_)
    atts = e_ / jnp.sum(e_, axis=-1, keepdims=True)
    comp_emb2 = (key_emb @ comp_W.T + comp_b)[..., 0]
    neigh_list = []
    for b in range(B_):
        att = atts[b]
        keep = _topk_out_edge_mask(src, att, TOPK_).astype(f32)
        msg = comp_emb2 * (att * keep)[:, None]
        neigh_list.append(jax.ops.segment_sum(msg, dst, num_segments=N_ENT_))
    neigh = jnp.stack(neigh_list, axis=0)
    mu = jnp.mean(neigh, axis=-1, keepdims=True)
    var = jnp.mean((neigh - mu) ** 2, axis=-1, keepdims=True)
    x = (neigh - mu) / jnp.sqrt(var + 1e-5)
    Q = x @ q_W.T + q_b
    Kp = x @ k_W.T + k_b
    sel_scale = H_ ** (-0.5)
    scores = jnp.matmul(Q, jnp.swapaxes(Kp, -2, -1)) / sel_scale
    scores = scores - jnp.max(scores, axis=-1, keepdims=True)
    attw = jax.nn.softmax(scores, axis=-1)
    importance = jnp.mean(attw, axis=1)
    _, topk_idx = jax.lax.top_k(importance, KSEL_)
    return _pallas_tail(x, attw, topk_idx, v_W, v_b, kg_W, kg_b)
